# baseline (speedup 1.0000x reference)
import jax
import jax.numpy as jnp
from jax.experimental import pallas as pl

_FB = 4096


def _dense_kernel(bins_ref, tt_ref, logits_ref, out_ref):
    b = bins_ref[0, :]
    nb = 32.0
    lw32 = jnp.log(b[1:] - b[:32])
    lw128 = jnp.concatenate([lw32, lw32, lw32, lw32])

    lane = jax.lax.broadcasted_iota(jnp.int32, (1, 128), 1)
    kconst = (lane % 32).astype(jnp.float32)
    grp = lane // 32

    w4 = (jax.lax.broadcasted_iota(jnp.int32, (4, 128), 0) == grp).astype(
        jnp.float32)
    g4 = (jax.lax.broadcasted_iota(jnp.int32, (128, 4), 1)
          == grp.reshape(128, 1)).astype(jnp.float32)

    tt = tt_ref[0]
    idx_t = jnp.clip(jnp.ceil(tt * nb) - 1.0, 0.0, nb - 1.0)

    idx_big = jax.lax.dot_general(
        idx_t, w4, (((0,), (0,)), ((), ())),
        preferred_element_type=jnp.float32)

    x = logits_ref[...]
    m = idx_big == kconst
    e = jnp.exp(x)
    xs = jnp.where(m, x - lw128[None, :], 0.0)
    st = jax.lax.dot_general(
        g4, e, (((0,), (1,)), ((), ())),
        preferred_element_type=jnp.float32)
    gxt = jax.lax.dot_general(
        g4, xs, (((0,), (1,)), ((), ())),
        preferred_element_type=jnp.float32,
        precision=jax.lax.Precision.HIGHEST)
    out_ref[0] = jnp.log(st) - gxt


@jax.jit
def kernel(targets, logits, bins):
    R, C = targets.shape
    nb = bins.shape[0]
    nflat = R * C * 32 // 128
    nblk = nflat // _FB

    l2 = logits.reshape(nflat, 128)
    tt = targets.reshape(nblk, _FB, 4).transpose(0, 2, 1)

    out_t = pl.pallas_call(
        _dense_kernel,
        grid=(nblk,),
        in_specs=[
            pl.BlockSpec((1, nb), lambda i: (0, 0)),
            pl.BlockSpec((1, 4, _FB), lambda i: (i, 0, 0)),
            pl.BlockSpec((_FB, 128), lambda i: (i, 0)),
        ],
        out_specs=pl.BlockSpec((1, 4, _FB), lambda i: (i, 0, 0)),
        out_shape=jax.ShapeDtypeStruct((nblk, 4, _FB), jnp.float32),
    )(bins.reshape(1, nb), tt, l2)

    return out_t.transpose(0, 2, 1).reshape(R, C)


# speedup vs baseline: 1.0590x; 1.0590x over previous
import jax
import jax.numpy as jnp
from jax import lax
from jax.experimental import pallas as pl
from jax.experimental.pallas import tpu as pltpu
from jax.experimental.pallas import tpu_sc as plsc

_FB = 4096
_NW = 32
_CHUNK = 4096
_GW = 128
_NJ = _CHUNK // _GW


def _lse_kernel(bins_ref, logits_ref, out_ref):
    lane = jax.lax.broadcasted_iota(jnp.int32, (1, 128), 1)
    grp = lane // 32
    g4 = (jax.lax.broadcasted_iota(jnp.int32, (128, 4), 1)
          == grp.reshape(128, 1)).astype(jnp.float32)
    e = jnp.exp(logits_ref[...])
    st = jax.lax.dot_general(
        g4, e, (((0,), (1,)), ((), ())),
        preferred_element_type=jnp.float32)
    lwc = jnp.log(bins_ref[0, 1] - bins_ref[0, 0])
    out_ref[0] = jnp.log(st) + lwc


def _sc_combine(t_hbm, logits_hbm, lset_hbm, out_hbm,
                t_v, offs_v, lfo_v, g_v, lse_v, out_v, sem):
    n_total = out_hbm.shape[0]
    per_w = n_total // _NW
    nchunks = per_w // _CHUNK
    wid = lax.axis_index("s") * 2 + lax.axis_index("c")
    iota = lax.iota(jnp.int32, 16)

    def chunk_body(c, carry):
        base = wid * per_w + c * _CHUNK
        pltpu.sync_copy(t_hbm.at[pl.ds(base, _CHUNK)], t_v)

        def comp_body(j, carry2):
            for p in range(8):
                s = j * 128 + p * 16
                t16 = t_v[pl.ds(s, 16)]
                y = t16 * 32.0
                yi = y.astype(jnp.int32)
                yf = yi.astype(jnp.float32)
                idx = jnp.where(y > yf, yi, yi - 1)
                idx = jnp.clip(idx, 0, 31)
                n16 = base + s + iota
                offs_v[j, pl.ds(p * 16, 16)] = n16 * 32 + idx
                lfo_v[j, pl.ds(p * 16, 16)] = (
                    ((n16 >> 14) << 14) + ((n16 & 3) << 12)
                    + ((n16 & 16383) >> 2))
            return carry2

        lax.fori_loop(0, _NJ, comp_body, 0)

        dmas = []
        for j in range(_NJ):
            dmas.append(pltpu.async_copy(
                logits_hbm.at[offs_v.at[j]], g_v.at[pl.ds(j * _GW, _GW)],
                sem))
            dmas.append(pltpu.async_copy(
                lset_hbm.at[lfo_v.at[j]], lse_v.at[pl.ds(j * _GW, _GW)],
                sem))
        for d in dmas:
            d.wait()

        def fin_body(i, carry3):
            sl = pl.ds(i * 16, 16)
            out_v[sl] = lse_v[sl] - g_v[sl]
            return carry3

        lax.fori_loop(0, _CHUNK // 16, fin_body, 0)
        pltpu.sync_copy(out_v, out_hbm.at[pl.ds(base, _CHUNK)])
        return carry

    lax.fori_loop(0, nchunks, chunk_body, 0)


@jax.jit
def kernel(targets, logits, bins):
    R, C = targets.shape
    nflat = R * C * 32 // 128
    nblk = nflat // _FB
    ntar = R * C

    l2 = logits.reshape(nflat, 128)
    lse_t = pl.pallas_call(
        _lse_kernel,
        grid=(nblk,),
        in_specs=[
            pl.BlockSpec((1, bins.shape[0]), lambda i: (0, 0)),
            pl.BlockSpec((_FB, 128), lambda i: (i, 0)),
        ],
        out_specs=pl.BlockSpec((1, 4, _FB), lambda i: (i, 0, 0)),
        out_shape=jax.ShapeDtypeStruct((nblk, 4, _FB), jnp.float32),
    )(bins.reshape(1, bins.shape[0]), l2)

    mesh = plsc.VectorSubcoreMesh(core_axis_name="c", subcore_axis_name="s")
    sc = pl.kernel(
        _sc_combine,
        mesh=mesh,
        out_type=jax.ShapeDtypeStruct((ntar,), jnp.float32),
        scratch_types=[
            pltpu.VMEM((_CHUNK,), jnp.float32),
            pltpu.VMEM((_NJ, _GW), jnp.int32),
            pltpu.VMEM((_NJ, _GW), jnp.int32),
            pltpu.VMEM((_CHUNK,), jnp.float32),
            pltpu.VMEM((_CHUNK,), jnp.float32),
            pltpu.VMEM((_CHUNK,), jnp.float32),
            pltpu.SemaphoreType.DMA,
        ],
    )
    out_flat = sc(targets.reshape(ntar), logits.reshape(ntar * 32),
                  lse_t.reshape(ntar))
    return out_flat.reshape(R, C)
